# Initial kernel scaffold; baseline (speedup 1.0000x reference)
import jax
import jax.numpy as jnp
from jax.experimental import pallas as pl


def kernel(x, edge_index, W_in, b_in, W_res, b_res, W_rel, W_root, b_conv, gamma, beta, W_out, b_out):
    raise NotImplementedError("write your pallas kernel here")


# speedup vs baseline: 11.7864x; 11.7864x over previous
import functools

import jax
import jax.numpy as jnp
from jax import lax
from jax.experimental import pallas as pl
from jax.experimental.pallas import tpu as pltpu
from jax.experimental.pallas import tpu_sc as plsc

_N = 20736
_E = 414720
_H = 64
_NC = 2
_NS = 16
_NW = _NC * _NS
_CHUNK = 128
_Q0 = 76
_Q1 = 127
_TOTC = _NS * (_Q0 + _Q1)
_IDXROWS = _TOTC + 32
_EPAD = _IDXROWS * _CHUNK - _E
_KST = 64
_RPS = _N // _NS
_ZR = 54
_BLK = 2592


def _segment_sum_sc(h, src_r, dst_r):
    mesh = plsc.VectorSubcoreMesh(core_axis_name="c", subcore_axis_name="s")

    @functools.partial(
        pl.kernel,
        out_type=jax.ShapeDtypeStruct((_NC, _N, _H), jnp.float32),
        mesh=mesh,
        scratch_types=[
            pltpu.VMEM((_KST, _CHUNK), jnp.int32),
            pltpu.VMEM((_KST, _CHUNK), jnp.int32),
            pltpu.VMEM((_CHUNK, _H), jnp.float32),
            pltpu.VMEM((_CHUNK, _H), jnp.float32),
            pltpu.VMEM((_ZR, _H), jnp.float32),
            pltpu.VMEM_SHARED((_N + 8, _H), jnp.float32),
            pltpu.SemaphoreType.DMA,
            pltpu.SemaphoreType.DMA,
        ],
        compiler_params=pltpu.CompilerParams(use_tc_tiling_on_sc=False),
    )
    def seg_kernel(h_hbm, src_hbm, dst_hbm, out_hbm, src_v, dst_v,
                   buf0, buf1, zv, acc, sg0, sg1):
        c = lax.axis_index("c")
        s = lax.axis_index("s")
        q = jnp.where(c == 0, _Q0, _Q1)
        base = jnp.where(c == 0, s * _Q0, _NS * _Q0 + s * _Q1)

        @pl.loop(0, _ZR)
        def _(r):
            @pl.loop(0, _H, step=16)
            def _(c0):
                zv[r, pl.ds(c0, 16)] = jnp.zeros((16,), jnp.float32)

        @pl.loop(0, _RPS, step=_ZR)
        def _(r0):
            pltpu.sync_copy(zv, acc.at[pl.ds(s * _RPS + r0, _ZR)])

        @pl.when(s == 0)
        def _():
            pltpu.sync_copy(zv.at[pl.ds(0, 8)], acc.at[pl.ds(_N, 8)])

        plsc.subcore_barrier()

        for b in range(2):
            nb = jnp.minimum(q - b * _KST, _KST)
            pltpu.sync_copy(src_hbm.at[pl.ds(base + b * _KST, _KST)], src_v)
            pltpu.sync_copy(dst_hbm.at[pl.ds(base + b * _KST, _KST)], dst_v)

            pltpu.async_copy(h_hbm.at[src_v.at[0]], buf0, sg0)

            @pl.when(1 < nb)
            def _():
                pltpu.async_copy(h_hbm.at[src_v.at[1]], buf1, sg1)

            @pl.loop(0, _KST, step=2)
            def _(j):
                @pl.when(j < nb)
                def _():
                    pltpu.make_async_copy(h_hbm.at[src_v.at[0]], buf0, sg0).wait()
                    pltpu.sync_copy(buf0, acc.at[dst_v.at[j]], add=True)

                @pl.when(j + 2 < nb)
                def _():
                    pltpu.async_copy(h_hbm.at[src_v.at[j + 2]], buf0, sg0)

                @pl.when(j + 1 < nb)
                def _():
                    pltpu.make_async_copy(h_hbm.at[src_v.at[0]], buf1, sg1).wait()
                    pltpu.sync_copy(buf1, acc.at[dst_v.at[j + 1]], add=True)

                @pl.when(j + 3 < nb)
                def _():
                    pltpu.async_copy(h_hbm.at[src_v.at[j + 3]], buf1, sg1)

        plsc.subcore_barrier()
        pltpu.sync_copy(acc.at[pl.ds(s * _RPS, _RPS)],
                        out_hbm.at[c, pl.ds(s * _RPS, _RPS)])

    return seg_kernel(h, src_r, dst_r)


def _in_proj_tc(x, W_in, b_in, W_res, b_res):
    def body(x_ref, wi_ref, bi_ref, wr_ref, br_ref, h_ref, r_ref):
        h = jnp.dot(x_ref[...], wi_ref[...],
                    preferred_element_type=jnp.float32) + bi_ref[...]
        h_ref[...] = h
        r_ref[...] = jnp.dot(h, wr_ref[...],
                             preferred_element_type=jnp.float32) + br_ref[...]

    return pl.pallas_call(
        body,
        grid=(_N // _BLK,),
        in_specs=[
            pl.BlockSpec((_BLK, 9), lambda i: (i, 0)),
            pl.BlockSpec((9, _H), lambda i: (0, 0)),
            pl.BlockSpec((1, _H), lambda i: (0, 0)),
            pl.BlockSpec((_H, _H), lambda i: (0, 0)),
            pl.BlockSpec((1, _H), lambda i: (0, 0)),
        ],
        out_specs=[pl.BlockSpec((_BLK, _H), lambda i: (i, 0)),
                   pl.BlockSpec((_BLK, _H), lambda i: (i, 0))],
        out_shape=[jax.ShapeDtypeStruct((_N, _H), jnp.float32),
                   jax.ShapeDtypeStruct((_N, _H), jnp.float32)],
    )(x, W_in, b_in.reshape(1, _H), W_res, b_res.reshape(1, _H))


def _layer_tc(part, h, res, W_cat, bias, gamma2, beta2):
    def body(p_ref, h_ref, r_ref, w_ref, b_ref, g_ref, be_ref, o_ref):
        agg = p_ref[0] + p_ref[1]
        t = (jnp.dot(agg, w_ref[0:_H], preferred_element_type=jnp.float32)
             + jnp.dot(h_ref[...], w_ref[_H:2 * _H],
                       preferred_element_type=jnp.float32)
             + b_ref[...])
        mu = jnp.mean(t, axis=-1, keepdims=True)
        d = t - mu
        var = jnp.mean(d * d, axis=-1, keepdims=True)
        y = d * lax.rsqrt(var + 1e-5) * g_ref[...] + be_ref[...]
        y = jnp.where(y >= 0, y, 0.01 * y)
        o_ref[...] = y + r_ref[...]

    return pl.pallas_call(
        body,
        grid=(_N // _BLK,),
        in_specs=[
            pl.BlockSpec((_NC, _BLK, _H), lambda i: (0, i, 0)),
            pl.BlockSpec((_BLK, _H), lambda i: (i, 0)),
            pl.BlockSpec((_BLK, _H), lambda i: (i, 0)),
            pl.BlockSpec((2 * _H, _H), lambda i: (0, 0)),
            pl.BlockSpec((1, _H), lambda i: (0, 0)),
            pl.BlockSpec((1, _H), lambda i: (0, 0)),
            pl.BlockSpec((1, _H), lambda i: (0, 0)),
        ],
        out_specs=pl.BlockSpec((_BLK, _H), lambda i: (i, 0)),
        out_shape=jax.ShapeDtypeStruct((_N, _H), jnp.float32),
    )(part, h, res, W_cat, bias, gamma2, beta2)


def _out_proj_tc(h, W_out, b_out):
    def body(h_ref, w_ref, b_ref, o_ref):
        o_ref[...] = jnp.dot(h_ref[...], w_ref[...],
                             preferred_element_type=jnp.float32) + b_ref[...]

    return pl.pallas_call(
        body,
        grid=(_N // _BLK,),
        in_specs=[
            pl.BlockSpec((_BLK, _H), lambda i: (i, 0)),
            pl.BlockSpec((_H, 9), lambda i: (0, 0)),
            pl.BlockSpec((1, 9), lambda i: (0, 0)),
        ],
        out_specs=pl.BlockSpec((_BLK, 9), lambda i: (i, 0)),
        out_shape=jax.ShapeDtypeStruct((_N, 9), jnp.float32),
    )(h, W_out, b_out.reshape(1, 9))


def kernel(x, edge_index, W_in, b_in, W_res, b_res, W_rel, W_root, b_conv,
           gamma, beta, W_out, b_out):
    src_r = jnp.concatenate(
        [edge_index[0], jnp.zeros((_EPAD,), jnp.int32)]
    ).reshape(_IDXROWS, _CHUNK)
    dst_r = jnp.concatenate(
        [edge_index[1], jnp.full((_EPAD,), _N, jnp.int32)]
    ).reshape(_IDXROWS, _CHUNK)
    gamma2 = gamma.reshape(1, _H)
    beta2 = beta.reshape(1, _H)

    h, res = _in_proj_tc(x, W_in, b_in, W_res, b_res)
    for l in range(4):
        part = _segment_sum_sc(h, src_r, dst_r)
        W_cat = jnp.concatenate([W_rel[l], W_root[l]], axis=0)
        h = _layer_tc(part, h, res, W_cat, b_conv[l].reshape(1, _H),
                      gamma2, beta2)
    out = _out_proj_tc(h, W_out, b_out)
    return out.reshape(_N // 81, 9, 9, 9)


# speedup vs baseline: 13.6441x; 1.1576x over previous
import functools

import jax
import jax.numpy as jnp
from jax import lax
from jax.experimental import pallas as pl
from jax.experimental.pallas import tpu as pltpu
from jax.experimental.pallas import tpu_sc as plsc

_N = 20736
_E = 414720
_H = 64
_NC = 2
_NS = 16
_NW = _NC * _NS
_CHUNK = 128
_Q0 = 76
_Q1 = 127
_TOTC = _NS * (_Q0 + _Q1)
_IDXROWS = _TOTC + 32
_EPAD = _IDXROWS * _CHUNK - _E
_KST = 64
_RPS = _N // _NS
_ZR = 54
_BLK = 2592


def _segment_sum_sc(h, src_r, dst_r):
    mesh = plsc.VectorSubcoreMesh(core_axis_name="c", subcore_axis_name="s")

    @functools.partial(
        pl.kernel,
        out_type=jax.ShapeDtypeStruct((_NC, _N, _H), jnp.float32),
        mesh=mesh,
        scratch_types=[
            pltpu.VMEM((_KST, _CHUNK), jnp.int32),
            pltpu.VMEM((_KST, _CHUNK), jnp.int32),
            pltpu.VMEM((_CHUNK, _H), jnp.float32),
            pltpu.VMEM((_CHUNK, _H), jnp.float32),
            pltpu.VMEM((_ZR, _H), jnp.float32),
            pltpu.VMEM_SHARED((_N + 8, _H), jnp.float32),
            pltpu.SemaphoreType.DMA,
            pltpu.SemaphoreType.DMA,
        ],
        compiler_params=pltpu.CompilerParams(use_tc_tiling_on_sc=False),
    )
    def seg_kernel(h_hbm, src_hbm, dst_hbm, out_hbm, src_v, dst_v,
                   buf0, buf1, zv, acc, sg0, sg1):
        c = lax.axis_index("c")
        s = lax.axis_index("s")
        q = jnp.where(c == 0, _Q0, _Q1)
        base = jnp.where(c == 0, s * _Q0, _NS * _Q0 + s * _Q1)

        @pl.loop(0, _ZR)
        def _(r):
            @pl.loop(0, _H, step=16)
            def _(c0):
                zv[r, pl.ds(c0, 16)] = jnp.zeros((16,), jnp.float32)

        @pl.loop(0, _RPS, step=_ZR)
        def _(r0):
            pltpu.sync_copy(zv, acc.at[pl.ds(s * _RPS + r0, _ZR)])

        @pl.when(s == 0)
        def _():
            pltpu.sync_copy(zv.at[pl.ds(0, 8)], acc.at[pl.ds(_N, 8)])

        plsc.subcore_barrier()

        for b in range(2):
            nb = jnp.minimum(q - b * _KST, _KST)
            pltpu.sync_copy(src_hbm.at[pl.ds(base + b * _KST, _KST)], src_v)
            pltpu.sync_copy(dst_hbm.at[pl.ds(base + b * _KST, _KST)], dst_v)

            pltpu.async_copy(h_hbm.at[src_v.at[0]], buf0, sg0)

            @pl.when(1 < nb)
            def _():
                pltpu.async_copy(h_hbm.at[src_v.at[1]], buf1, sg1)

            @pl.loop(0, _KST, step=2)
            def _(j):
                @pl.when(j < nb)
                def _():
                    pltpu.make_async_copy(h_hbm.at[src_v.at[0]], buf0, sg0).wait()
                    pltpu.sync_copy(buf0, acc.at[dst_v.at[j]], add=True)

                @pl.when(j + 2 < nb)
                def _():
                    pltpu.async_copy(h_hbm.at[src_v.at[j + 2]], buf0, sg0)

                @pl.when(j + 1 < nb)
                def _():
                    pltpu.make_async_copy(h_hbm.at[src_v.at[0]], buf1, sg1).wait()
                    pltpu.sync_copy(buf1, acc.at[dst_v.at[j + 1]], add=True)

                @pl.when(j + 3 < nb)
                def _():
                    pltpu.async_copy(h_hbm.at[src_v.at[j + 3]], buf1, sg1)

        plsc.subcore_barrier()
        pltpu.sync_copy(acc.at[pl.ds(s * _RPS, _RPS)],
                        out_hbm.at[c, pl.ds(s * _RPS, _RPS)])

    return seg_kernel(h, src_r, dst_r)


_N2 = _N // 2
_H2 = 2 * _H
_BLK2 = _BLK // 2


def _bd(W):
    z = jnp.zeros_like(W)
    return jnp.concatenate(
        [jnp.concatenate([W, z], axis=1), jnp.concatenate([z, W], axis=1)],
        axis=0)


def _tile2(v):
    return jnp.tile(v, 2).reshape(1, -1)


def _in_proj_tc(x2, Wi2, bi2, Wr2, br2):
    def body(x_ref, wi_ref, bi_ref, wr_ref, br_ref, h_ref, r_ref):
        h = jnp.dot(x_ref[...], wi_ref[...],
                    preferred_element_type=jnp.float32) + bi_ref[...]
        h_ref[...] = h
        r_ref[...] = jnp.dot(h, wr_ref[...],
                             preferred_element_type=jnp.float32) + br_ref[...]

    return pl.pallas_call(
        body,
        grid=(_N2 // _BLK2,),
        in_specs=[
            pl.BlockSpec((_BLK2, 18), lambda i: (i, 0)),
            pl.BlockSpec((18, _H2), lambda i: (0, 0)),
            pl.BlockSpec((1, _H2), lambda i: (0, 0)),
            pl.BlockSpec((_H2, _H2), lambda i: (0, 0)),
            pl.BlockSpec((1, _H2), lambda i: (0, 0)),
        ],
        out_specs=[pl.BlockSpec((_BLK2, _H2), lambda i: (i, 0)),
                   pl.BlockSpec((_BLK2, _H2), lambda i: (i, 0))],
        out_shape=[jax.ShapeDtypeStruct((_N2, _H2), jnp.float32),
                   jax.ShapeDtypeStruct((_N2, _H2), jnp.float32)],
    )(x2, Wi2, bi2, Wr2, br2)


def _layer_tc(part2, h2, res2, Wc2, bias2, g2, be2, Mavg):
    def body(p_ref, h_ref, r_ref, w_ref, b_ref, g_ref, be_ref, m_ref, o_ref):
        agg = p_ref[0] + p_ref[1]
        t = (jnp.dot(agg, w_ref[0:_H2], preferred_element_type=jnp.float32)
             + jnp.dot(h_ref[...], w_ref[_H2:2 * _H2],
                       preferred_element_type=jnp.float32)
             + b_ref[...])
        mu = jnp.dot(t, m_ref[...], preferred_element_type=jnp.float32,
                     precision=lax.Precision.HIGHEST)
        d = t - mu
        var = jnp.dot(d * d, m_ref[...], preferred_element_type=jnp.float32,
                      precision=lax.Precision.HIGHEST)
        y = d * lax.rsqrt(var + 1e-5) * g_ref[...] + be_ref[...]
        y = jnp.where(y >= 0, y, 0.01 * y)
        o_ref[...] = y + r_ref[...]

    return pl.pallas_call(
        body,
        grid=(_N2 // _BLK2,),
        in_specs=[
            pl.BlockSpec((_NC, _BLK2, _H2), lambda i: (0, i, 0)),
            pl.BlockSpec((_BLK2, _H2), lambda i: (i, 0)),
            pl.BlockSpec((_BLK2, _H2), lambda i: (i, 0)),
            pl.BlockSpec((2 * _H2, _H2), lambda i: (0, 0)),
            pl.BlockSpec((1, _H2), lambda i: (0, 0)),
            pl.BlockSpec((1, _H2), lambda i: (0, 0)),
            pl.BlockSpec((1, _H2), lambda i: (0, 0)),
            pl.BlockSpec((_H2, _H2), lambda i: (0, 0)),
        ],
        out_specs=pl.BlockSpec((_BLK2, _H2), lambda i: (i, 0)),
        out_shape=jax.ShapeDtypeStruct((_N2, _H2), jnp.float32),
    )(part2, h2, res2, Wc2, bias2, g2, be2, Mavg)


def _out_proj_tc(h2, Wo2, bo2):
    def body(h_ref, w_ref, b_ref, o_ref):
        o_ref[...] = jnp.dot(h_ref[...], w_ref[...],
                             preferred_element_type=jnp.float32) + b_ref[...]

    return pl.pallas_call(
        body,
        grid=(_N2 // _BLK2,),
        in_specs=[
            pl.BlockSpec((_BLK2, _H2), lambda i: (i, 0)),
            pl.BlockSpec((_H2, 18), lambda i: (0, 0)),
            pl.BlockSpec((1, 18), lambda i: (0, 0)),
        ],
        out_specs=pl.BlockSpec((_BLK2, 18), lambda i: (i, 0)),
        out_shape=jax.ShapeDtypeStruct((_N2, 18), jnp.float32),
    )(h2, Wo2, bo2)


def kernel(x, edge_index, W_in, b_in, W_res, b_res, W_rel, W_root, b_conv,
           gamma, beta, W_out, b_out):
    src_r = jnp.concatenate(
        [edge_index[0], jnp.zeros((_EPAD,), jnp.int32)]
    ).reshape(_IDXROWS, _CHUNK)
    dst_r = jnp.concatenate(
        [edge_index[1], jnp.full((_EPAD,), _N, jnp.int32)]
    ).reshape(_IDXROWS, _CHUNK)

    x2 = x.reshape(_N2, 18)
    g2 = _tile2(gamma)
    be2 = _tile2(beta)
    Mavg = _bd(jnp.full((_H, _H), 1.0 / _H, jnp.float32))

    h2, res2 = _in_proj_tc(x2, _bd(W_in), _tile2(b_in), _bd(W_res),
                           _tile2(b_res))
    for l in range(4):
        part = _segment_sum_sc(h2.reshape(_N, _H), src_r, dst_r)
        Wc2 = jnp.concatenate([_bd(W_rel[l]), _bd(W_root[l])], axis=0)
        h2 = _layer_tc(part.reshape(_NC, _N2, _H2), h2, res2, Wc2,
                       _tile2(b_conv[l]), g2, be2, Mavg)
    out2 = _out_proj_tc(h2, _bd(W_out), _tile2(b_out))
    return out2.reshape(_N // 81, 9, 9, 9)


# speedup vs baseline: 14.8377x; 1.0875x over previous
import functools

import jax
import jax.numpy as jnp
from jax import lax
from jax.experimental import pallas as pl
from jax.experimental.pallas import tpu as pltpu
from jax.experimental.pallas import tpu_sc as plsc

_N = 20736
_E = 414720
_H = 64
_NC = 2
_NS = 16
_NW = _NC * _NS
_CHUNK = 128
_Q0 = 104
_Q1 = 99
_TOTC = _NS * (_Q0 + _Q1)
_IDXROWS = _TOTC + 32
_EPAD = _IDXROWS * _CHUNK - _E
_KST = 64
_RPS = _N // _NS
_ZR = 54
_BLK = 2592


def _segment_sum_sc(h, src_r, dst_r):
    mesh = plsc.VectorSubcoreMesh(core_axis_name="c", subcore_axis_name="s")

    @functools.partial(
        pl.kernel,
        out_type=jax.ShapeDtypeStruct((_NC, _N, _H), jnp.float32),
        mesh=mesh,
        scratch_types=[
            pltpu.VMEM((_KST, _CHUNK), jnp.int32),
            pltpu.VMEM((_KST, _CHUNK), jnp.int32),
            pltpu.VMEM((_CHUNK, _H), jnp.float32),
            pltpu.VMEM((_CHUNK, _H), jnp.float32),
            pltpu.VMEM((_ZR, _H), jnp.float32),
            pltpu.VMEM_SHARED((_N + 8, _H), jnp.float32),
            pltpu.SemaphoreType.DMA,
            pltpu.SemaphoreType.DMA,
        ],
        compiler_params=pltpu.CompilerParams(use_tc_tiling_on_sc=False),
    )
    def seg_kernel(h_hbm, src_hbm, dst_hbm, out_hbm, src_v, dst_v,
                   buf0, buf1, zv, acc, sg0, sg1):
        c = lax.axis_index("c")
        s = lax.axis_index("s")
        q = jnp.where(c == 0, _Q0, _Q1)
        base = jnp.where(c == 0, s * _Q0, _NS * _Q0 + s * _Q1)

        @pl.loop(0, _ZR)
        def _(r):
            @pl.loop(0, _H, step=16)
            def _(c0):
                zv[r, pl.ds(c0, 16)] = jnp.zeros((16,), jnp.float32)

        @pl.loop(0, _RPS, step=_ZR)
        def _(r0):
            pltpu.sync_copy(zv, acc.at[pl.ds(s * _RPS + r0, _ZR)])

        @pl.when(s == 0)
        def _():
            pltpu.sync_copy(zv.at[pl.ds(0, 8)], acc.at[pl.ds(_N, 8)])

        plsc.subcore_barrier()

        for b in range(2):
            nb = jnp.minimum(q - b * _KST, _KST)
            pltpu.sync_copy(src_hbm.at[pl.ds(base + b * _KST, _KST)], src_v)
            pltpu.sync_copy(dst_hbm.at[pl.ds(base + b * _KST, _KST)], dst_v)

            pltpu.async_copy(h_hbm.at[src_v.at[0]], buf0, sg0)

            @pl.when(1 < nb)
            def _():
                pltpu.async_copy(h_hbm.at[src_v.at[1]], buf1, sg1)

            @pl.loop(0, _KST, step=2)
            def _(j):
                @pl.when(j < nb)
                def _():
                    pltpu.make_async_copy(h_hbm.at[src_v.at[0]], buf0, sg0).wait()
                    pltpu.sync_copy(buf0, acc.at[dst_v.at[j]], add=True)

                @pl.when(j + 2 < nb)
                def _():
                    pltpu.async_copy(h_hbm.at[src_v.at[j + 2]], buf0, sg0)

                @pl.when(j + 1 < nb)
                def _():
                    pltpu.make_async_copy(h_hbm.at[src_v.at[0]], buf1, sg1).wait()
                    pltpu.sync_copy(buf1, acc.at[dst_v.at[j + 1]], add=True)

                @pl.when(j + 3 < nb)
                def _():
                    pltpu.async_copy(h_hbm.at[src_v.at[j + 3]], buf1, sg1)

        plsc.subcore_barrier()
        pltpu.sync_copy(acc.at[pl.ds(s * _RPS, _RPS)],
                        out_hbm.at[c, pl.ds(s * _RPS, _RPS)])

    return seg_kernel(h, src_r, dst_r)


_N2 = _N // 2
_H2 = 2 * _H
_BLK2 = _BLK // 2


def _bd(W):
    z = jnp.zeros_like(W)
    return jnp.concatenate(
        [jnp.concatenate([W, z], axis=1), jnp.concatenate([z, W], axis=1)],
        axis=0)


def _tile2(v):
    return jnp.tile(v, 2).reshape(1, -1)


def _in_proj_tc(x2, Wi2, bi2, Wr2, br2):
    def body(x_ref, wi_ref, bi_ref, wr_ref, br_ref, h_ref, r_ref):
        h = jnp.dot(x_ref[...], wi_ref[...],
                    preferred_element_type=jnp.float32) + bi_ref[...]
        h_ref[...] = h
        r_ref[...] = jnp.dot(h, wr_ref[...],
                             preferred_element_type=jnp.float32) + br_ref[...]

    return pl.pallas_call(
        body,
        grid=(_N2 // _BLK2,),
        in_specs=[
            pl.BlockSpec((_BLK2, 18), lambda i: (i, 0)),
            pl.BlockSpec((18, _H2), lambda i: (0, 0)),
            pl.BlockSpec((1, _H2), lambda i: (0, 0)),
            pl.BlockSpec((_H2, _H2), lambda i: (0, 0)),
            pl.BlockSpec((1, _H2), lambda i: (0, 0)),
        ],
        out_specs=[pl.BlockSpec((_BLK2, _H2), lambda i: (i, 0)),
                   pl.BlockSpec((_BLK2, _H2), lambda i: (i, 0))],
        out_shape=[jax.ShapeDtypeStruct((_N2, _H2), jnp.float32),
                   jax.ShapeDtypeStruct((_N2, _H2), jnp.float32)],
    )(x2, Wi2, bi2, Wr2, br2)


def _layer_tc(part2, h2, res2, Wc2, bias2, g2, be2, Mavg):
    def body(p_ref, h_ref, r_ref, w_ref, b_ref, g_ref, be_ref, m_ref, o_ref):
        agg = p_ref[0] + p_ref[1]
        t = (jnp.dot(agg, w_ref[0:_H2], preferred_element_type=jnp.float32)
             + jnp.dot(h_ref[...], w_ref[_H2:2 * _H2],
                       preferred_element_type=jnp.float32)
             + b_ref[...])
        mu = jnp.dot(t, m_ref[...], preferred_element_type=jnp.float32,
                     precision=lax.Precision.HIGHEST)
        d = t - mu
        var = jnp.dot(d * d, m_ref[...], preferred_element_type=jnp.float32,
                      precision=lax.Precision.HIGHEST)
        y = d * lax.rsqrt(var + 1e-5) * g_ref[...] + be_ref[...]
        y = jnp.where(y >= 0, y, 0.01 * y)
        o_ref[...] = y + r_ref[...]

    return pl.pallas_call(
        body,
        grid=(_N2 // _BLK2,),
        in_specs=[
            pl.BlockSpec((_NC, _BLK2, _H2), lambda i: (0, i, 0)),
            pl.BlockSpec((_BLK2, _H2), lambda i: (i, 0)),
            pl.BlockSpec((_BLK2, _H2), lambda i: (i, 0)),
            pl.BlockSpec((2 * _H2, _H2), lambda i: (0, 0)),
            pl.BlockSpec((1, _H2), lambda i: (0, 0)),
            pl.BlockSpec((1, _H2), lambda i: (0, 0)),
            pl.BlockSpec((1, _H2), lambda i: (0, 0)),
            pl.BlockSpec((_H2, _H2), lambda i: (0, 0)),
        ],
        out_specs=pl.BlockSpec((_BLK2, _H2), lambda i: (i, 0)),
        out_shape=jax.ShapeDtypeStruct((_N2, _H2), jnp.float32),
    )(part2, h2, res2, Wc2, bias2, g2, be2, Mavg)


def _out_proj_tc(h2, Wo2, bo2):
    def body(h_ref, w_ref, b_ref, o_ref):
        o_ref[...] = jnp.dot(h_ref[...], w_ref[...],
                             preferred_element_type=jnp.float32) + b_ref[...]

    return pl.pallas_call(
        body,
        grid=(_N2 // _BLK2,),
        in_specs=[
            pl.BlockSpec((_BLK2, _H2), lambda i: (i, 0)),
            pl.BlockSpec((_H2, 18), lambda i: (0, 0)),
            pl.BlockSpec((1, 18), lambda i: (0, 0)),
        ],
        out_specs=pl.BlockSpec((_BLK2, 18), lambda i: (i, 0)),
        out_shape=jax.ShapeDtypeStruct((_N2, 18), jnp.float32),
    )(h2, Wo2, bo2)


def kernel(x, edge_index, W_in, b_in, W_res, b_res, W_rel, W_root, b_conv,
           gamma, beta, W_out, b_out):
    src_r = jnp.concatenate(
        [edge_index[0], jnp.zeros((_EPAD,), jnp.int32)]
    ).reshape(_IDXROWS, _CHUNK)
    dst_r = jnp.concatenate(
        [edge_index[1], jnp.full((_EPAD,), _N, jnp.int32)]
    ).reshape(_IDXROWS, _CHUNK)

    x2 = x.reshape(_N2, 18)
    g2 = _tile2(gamma)
    be2 = _tile2(beta)
    Mavg = _bd(jnp.full((_H, _H), 1.0 / _H, jnp.float32))

    h2, res2 = _in_proj_tc(x2, _bd(W_in), _tile2(b_in), _bd(W_res),
                           _tile2(b_res))
    for l in range(4):
        part = _segment_sum_sc(h2.reshape(_N, _H), src_r, dst_r)
        Wc2 = jnp.concatenate([_bd(W_rel[l]), _bd(W_root[l])], axis=0)
        h2 = _layer_tc(part.reshape(_NC, _N2, _H2), h2, res2, Wc2,
                       _tile2(b_conv[l]), g2, be2, Mavg)
    out2 = _out_proj_tc(h2, _bd(W_out), _tile2(b_out))
    return out2.reshape(_N // 81, 9, 9, 9)
